# baseline (speedup 1.0000x reference)
import dataclasses
import functools

import jax
import jax.numpy as jnp
import numpy as np
from jax import lax
from jax.experimental import pallas as pl
from jax.experimental.pallas import tpu as pltpu
from jax.experimental.pallas import tpu_sc as plsc

N = 10000
E = 160000
H = 128
TD = 8
L = 3
T = 16
NP = 10240
EP = 163840
EPC = EP // 32
KL = 160
NC = EPC // KL
ROWS_PER_TILE = NP // 16
CLIP = float(np.arctanh(np.float32(1.0 - 1e-5)))
BN = 2000
GRID = N // BN

f32 = jnp.float32
i32 = jnp.int32


def _clipnorm(v):
    nv = jnp.maximum(jnp.sqrt(jnp.sum(v * v, axis=-1, keepdims=True)), 1e-15)
    return v * jnp.minimum(1.0, CLIP / nv)


def _prep_body(nf_ref, w_ref, b_ref, o_ref):
    v = jnp.dot(nf_ref[...], w_ref[...], preferred_element_type=f32, precision=jax.lax.Precision.HIGHEST) + b_ref[...]
    o_ref[...] = _clipnorm(v)


def _prep(nf, w, b):
    return pl.pallas_call(
        _prep_body,
        grid=(GRID,),
        in_specs=[
            pl.BlockSpec((BN, H), lambda i: (i, 0)),
            pl.BlockSpec((H, H), lambda i: (0, 0)),
            pl.BlockSpec((1, H), lambda i: (0, 0)),
        ],
        out_specs=pl.BlockSpec((BN, H), lambda i: (i, 0)),
        out_shape=jax.ShapeDtypeStruct((N, H), f32),
    )(nf, w, b)


def _mm_body(t_ref, w_ref, b_ref, av_ref, m_ref, sd_ref):
    t = t_ref[...]
    m_ref[...] = jnp.dot(t, w_ref[...], preferred_element_type=f32, precision=jax.lax.Precision.HIGHEST) + b_ref[...]
    sd_ref[...] = jnp.dot(t, av_ref[...], preferred_element_type=f32, precision=jax.lax.Precision.HIGHEST)


def _mm(t, w, b, av):
    return pl.pallas_call(
        _mm_body,
        grid=(GRID,),
        in_specs=[
            pl.BlockSpec((BN, H), lambda i: (i, 0)),
            pl.BlockSpec((H, H), lambda i: (0, 0)),
            pl.BlockSpec((1, H), lambda i: (0, 0)),
            pl.BlockSpec((H, 8), lambda i: (0, 0)),
        ],
        out_specs=[
            pl.BlockSpec((BN, H), lambda i: (i, 0)),
            pl.BlockSpec((BN, 8), lambda i: (i, 0)),
        ],
        out_shape=[
            jax.ShapeDtypeStruct((N, H), f32),
            jax.ShapeDtypeStruct((N, 8), f32),
        ],
    )(t, w, b, av)


def _combine_body(a0_ref, a1_ref, s_ref, o_ref):
    r = 1.0 / (s_ref[...] + 1e-15)
    agg = (a0_ref[0] + a1_ref[0]) * r
    o_ref[...] = _clipnorm(jnp.maximum(agg, 0.0))


def _combine(aggp, s_col):
    return pl.pallas_call(
        _combine_body,
        grid=(GRID,),
        in_specs=[
            pl.BlockSpec((1, BN, H), lambda i: (0, i, 0)),
            pl.BlockSpec((1, BN, H), lambda i: (1, i, 0)),
            pl.BlockSpec((BN, 1), lambda i: (i, 0)),
        ],
        out_specs=pl.BlockSpec((BN, H), lambda i: (i, 0)),
        out_shape=jax.ShapeDtypeStruct((N, H), f32),
    )(aggp, aggp, s_col)


def _final_body(a0_ref, a1_ref, s_ref, s1_ref, s2_ref, wq_ref, wn1_ref, cn_ref,
                wn2_ref, we1s_ref, we1d_ref,
                x_ref, ns_ref, ea_ref, eb_ref):
    r = 1.0 / (s_ref[...] + 1e-15)
    agg = (a0_ref[0] + a1_ref[0]) * r
    v = jnp.maximum(agg, 0.0)
    nv = jnp.maximum(jnp.sqrt(jnp.sum(v * v, axis=-1, keepdims=True)), 1e-15)
    x_ref[...] = jnp.tanh(nv) * v / nv
    s3 = v * jnp.minimum(1.0, CLIP / nv)
    s1 = s1_ref[...]
    s2 = s2_ref[...]
    wq = wq_ref[...]
    eps = 1e-6

    def dlog(sn):
        ms = jnp.sqrt(jnp.mean(sn * sn, axis=-1, keepdims=True) + eps)
        return jnp.sum(sn * wq, axis=-1, keepdims=True) / ms

    d1, d2, d3 = dlog(s1), dlog(s2), dlog(s3)
    dm = jnp.maximum(jnp.maximum(d1, d2), d3)
    e1 = jnp.exp(d1 - dm)
    e2 = jnp.exp(d2 - dm)
    e3 = jnp.exp(d3 - dm)
    tang = (e1 * s1 + e2 * s2 + e3 * s3) / (e1 + e2 + e3)
    hid = jnp.maximum(
        jnp.dot(tang, wn1_ref[...], preferred_element_type=f32, precision=jax.lax.Precision.HIGHEST) + cn_ref[...], 0.0)
    ns_ref[...] = jnp.sum(hid * wn2_ref[...], axis=-1, keepdims=True)
    ea_ref[...] = jnp.dot(tang, we1s_ref[...], preferred_element_type=f32, precision=jax.lax.Precision.HIGHEST)
    eb_ref[...] = jnp.dot(tang, we1d_ref[...], preferred_element_type=f32, precision=jax.lax.Precision.HIGHEST)


def _final(aggp, s_col, s1, s2, wq, wn1a, cn, wn2, we1s, we1d):
    return pl.pallas_call(
        _final_body,
        grid=(GRID,),
        in_specs=[
            pl.BlockSpec((1, BN, H), lambda i: (0, i, 0)),
            pl.BlockSpec((1, BN, H), lambda i: (1, i, 0)),
            pl.BlockSpec((BN, 1), lambda i: (i, 0)),
            pl.BlockSpec((BN, H), lambda i: (i, 0)),
            pl.BlockSpec((BN, H), lambda i: (i, 0)),
            pl.BlockSpec((1, H), lambda i: (0, 0)),
            pl.BlockSpec((H, H), lambda i: (0, 0)),
            pl.BlockSpec((1, H), lambda i: (0, 0)),
            pl.BlockSpec((1, H), lambda i: (0, 0)),
            pl.BlockSpec((H, H), lambda i: (0, 0)),
            pl.BlockSpec((H, H), lambda i: (0, 0)),
        ],
        out_specs=[
            pl.BlockSpec((BN, H), lambda i: (i, 0)),
            pl.BlockSpec((BN, 1), lambda i: (i, 0)),
            pl.BlockSpec((BN, H), lambda i: (i, 0)),
            pl.BlockSpec((BN, H), lambda i: (i, 0)),
        ],
        out_shape=[
            jax.ShapeDtypeStruct((N, H), f32),
            jax.ShapeDtypeStruct((N, 1), f32),
            jax.ShapeDtypeStruct((N, H), f32),
            jax.ShapeDtypeStruct((N, H), f32),
        ],
    )(aggp, aggp, s_col, s1, s2, wq, wn1a, cn, wn2, we1s, we1d)


def _mesh():
    return plsc.VectorSubcoreMesh(core_axis_name="c", subcore_axis_name="s")


def _sc_params():
    cp = pltpu.CompilerParams()
    if "needs_layout_passes" in pltpu.CompilerParams.__dataclass_fields__:
        cp = dataclasses.replace(cp, needs_layout_passes=False)
    return cp


def _sc_layer(m, a_s, a_d_p, a_t, cst16, src_p, dst_p, et_p):
    @functools.partial(
        pl.kernel,
        out_type=[
            jax.ShapeDtypeStruct((2, NP, H), f32),
            jax.ShapeDtypeStruct((2, NP), f32),
        ],
        mesh=_mesh(),
        compiler_params=_sc_params(),
        scratch_types=[
            pltpu.VMEM((16,), f32),
            pltpu.VMEM((16,), f32),
            pltpu.VMEM((KL,), i32),
            pltpu.VMEM((KL,), i32),
            pltpu.VMEM((KL,), i32),
            pltpu.VMEM((KL,), i32),
            pltpu.VMEM((KL,), i32),
            pltpu.VMEM((KL,), i32),
            pltpu.VMEM((KL,), f32),
            pltpu.VMEM((KL,), f32),
            pltpu.VMEM((KL,), f32),
            pltpu.VMEM((KL,), f32),
            pltpu.VMEM((KL,), f32),
            pltpu.VMEM((KL,), f32),
            pltpu.VMEM((KL, H), f32),
            pltpu.VMEM((KL, H), f32),
            pltpu.VMEM_SHARED((NP, H), f32),
            pltpu.VMEM_SHARED((NP,), f32),
            pltpu.SemaphoreType.DMA,
            pltpu.SemaphoreType.DMA,
            pltpu.SemaphoreType.DMA,
            pltpu.SemaphoreType.DMA,
            pltpu.SemaphoreType.DMA,
            pltpu.SemaphoreType.DMA,
        ],
    )
    def k(m_hbm, as_hbm, ad_hbm, at_hbm, c_hbm, src_hbm, dst_hbm, et_hbm,
          agg_out, s_out,
          at_t, cst_t, srcva, dstva, etva, srcvb, dstvb, etvb,
          asga, adga, asgb, adgb, wva, wvb, rows_a, rows_b,
          agg_sh, s_sh, sema, semb, semsa, semsb, semia, semib):
        cid = lax.axis_index("c")
        sid = lax.axis_index("s")
        pltpu.sync_copy(at_hbm, at_t)
        pltpu.sync_copy(c_hbm, cst_t)
        zero = jnp.zeros((16,), f32)

        @pl.loop(0, KL)
        def _zr(rr):
            for fblk in range(8):
                rows_a[rr, pl.ds(fblk * 16, 16)] = zero

        @pl.loop(0, KL, step=16)
        def _zs(ii):
            wva[pl.ds(ii, 16)] = zero

        row0 = sid * ROWS_PER_TILE
        for kk in range(ROWS_PER_TILE // KL):
            pltpu.sync_copy(rows_a, agg_sh.at[pl.ds(row0 + kk * KL, KL)])
            pltpu.sync_copy(wva, s_sh.at[pl.ds(row0 + kk * KL, KL)])
        plsc.subcore_barrier()

        base = cid * (EP // 2) + sid * EPC

        def _start(c0, srcv, dstv, etv, asg, adg, rows, sem, semi):
            off = base + c0
            h1 = pltpu.async_copy(src_hbm.at[pl.ds(off, KL)], srcv, semi)
            h2 = pltpu.async_copy(dst_hbm.at[pl.ds(off, KL)], dstv, semi)
            h3 = pltpu.async_copy(et_hbm.at[pl.ds(off, KL)], etv, semi)
            h1.wait()
            h2.wait()
            h3.wait()
            ga = pltpu.async_copy(as_hbm.at[srcv], asg, sem)
            gb = pltpu.async_copy(ad_hbm.at[dstv], adg, sem)
            gc = pltpu.async_copy(m_hbm.at[srcv], rows, sem)
            return (ga, gb, gc)

        def _wait(hs):
            for h in hs:
                h.wait()

        def _process(dstv, etv, asg, adg, wv, rows):
            cv = cst_t[...]

            @pl.loop(0, KL, step=16)
            def _w(ii):
                a_sv = asg[pl.ds(ii, 16)]
                a_dv = adg[pl.ds(ii, 16)]
                ti = etv[pl.ds(ii, 16)]
                atg = plsc.load_gather(at_t, [ti])
                z = a_sv + a_dv + atg
                lr = jnp.maximum(z, 0.2 * z)
                adc = a_dv + cv
                ov = jnp.maximum(adc, 0.2 * adc)
                wvec = jnp.exp(lr - ov)
                wv[pl.ds(ii, 16)] = wvec
                for j in range(16):
                    w = wvec[j]
                    for fblk in range(8):
                        sl = pl.ds(fblk * 16, 16)
                        rows[ii + j, sl] = rows[ii + j, sl] * w

        def _scatter(dstv, wv, rows, sem):
            pltpu.async_copy(rows, agg_sh.at[dstv], sem, add=True).wait()
            pltpu.async_copy(wv, s_sh.at[dstv], sem, add=True).wait()
            return ()

        _start(0, srcva, dstva, etva, asga, adga, rows_a, sema, semia)

        @pl.loop(0, NC // 2)
        def _pair(i):
            c0 = 2 * KL * i
            hgb = _start(c0 + KL, srcvb, dstvb, etvb, asgb, adgb, rows_b,
                         semb, semib)
            pltpu.make_async_copy(as_hbm.at[srcva], asga, sema).wait()
            pltpu.make_async_copy(ad_hbm.at[dstva], adga, sema).wait()
            pltpu.make_async_copy(m_hbm.at[srcva], rows_a, sema).wait()
            _process(dstva, etva, asga, adga, wva, rows_a)
            hsa = _scatter(dstva, wva, rows_a, semsa)
            _wait(hgb)
            _process(dstvb, etvb, asgb, adgb, wvb, rows_b)
            hsb = _scatter(dstvb, wvb, rows_b, semsb)
            _wait(hsa)

            @pl.when(c0 + 2 * KL < EPC)
            def _():
                _start(c0 + 2 * KL, srcva, dstva, etva, asga, adga, rows_a,
                       sema, semia)

            _wait(hsb)

        plsc.subcore_barrier()
        sl = pl.ds(row0, ROWS_PER_TILE)
        pltpu.sync_copy(agg_sh.at[sl], agg_out.at[cid, sl])
        pltpu.sync_copy(s_sh.at[sl], s_out.at[cid, sl])

    return k(m, a_s, a_d_p, a_t, cst16, src_p, dst_p, et_p)


def _sc_edge(ea, eb_p, cte, w2, src_p, dst_p, et_p):
    @functools.partial(
        pl.kernel,
        out_type=jax.ShapeDtypeStruct((EP,), f32),
        mesh=_mesh(),
        compiler_params=_sc_params(),
        scratch_types=[
            pltpu.VMEM((T, H), f32),
            pltpu.VMEM((H,), f32),
            pltpu.VMEM((KL,), i32),
            pltpu.VMEM((KL,), i32),
            pltpu.VMEM((KL,), i32),
            pltpu.VMEM((KL,), i32),
            pltpu.VMEM((KL,), i32),
            pltpu.VMEM((KL,), i32),
            pltpu.VMEM((KL, H), f32),
            pltpu.VMEM((KL, H), f32),
            pltpu.VMEM((KL, H), f32),
            pltpu.VMEM((KL, H), f32),
            pltpu.VMEM((KL * 16,), f32),
            pltpu.VMEM((KL,), f32),
            pltpu.SemaphoreType.DMA,
            pltpu.SemaphoreType.DMA,
            pltpu.SemaphoreType.DMA,
            pltpu.SemaphoreType.DMA,
        ],
    )
    def k(ea_hbm, eb_hbm, cte_hbm, w2_hbm, src_hbm, dst_hbm, et_hbm, sc_out,
          cte_t, w2_t, srcva, dstva, etva, srcvb, dstvb, etvb,
          arows_a, brows_a, arows_b, brows_b, accs, scv, sema, semb,
          semia, semib):
        cid = lax.axis_index("c")
        sid = lax.axis_index("s")
        pltpu.sync_copy(cte_hbm, cte_t)
        pltpu.sync_copy(w2_hbm, w2_t)
        base = cid * (EP // 2) + sid * EPC
        lane = lax.iota(i32, 16)
        w2b = [w2_t[pl.ds(fblk * 16, 16)] for fblk in range(8)]

        def _start(c0, srcv, dstv, etv, arows, brows, sem, semi):
            off = base + c0
            h1 = pltpu.async_copy(src_hbm.at[pl.ds(off, KL)], srcv, semi)
            h2 = pltpu.async_copy(dst_hbm.at[pl.ds(off, KL)], dstv, semi)
            h3 = pltpu.async_copy(et_hbm.at[pl.ds(off, KL)], etv, semi)
            h1.wait()
            h2.wait()
            h3.wait()
            pltpu.async_copy(ea_hbm.at[srcv], arows, sem)
            pltpu.async_copy(eb_hbm.at[dstv], brows, sem)

        def _wait(srcv, dstv, arows, brows, sem):
            pltpu.make_async_copy(ea_hbm.at[srcv], arows, sem).wait()
            pltpu.make_async_copy(eb_hbm.at[dstv], brows, sem).wait()

        def _process(c0, etv, arows, brows):
            @pl.loop(0, KL, step=16)
            def _edge(ii):
                tvec = etv[pl.ds(ii, 16)]
                for j in range(16):
                    tj = tvec[j]
                    acc = jnp.zeros((16,), f32)
                    for fblk in range(8):
                        sl = pl.ds(fblk * 16, 16)
                        h = arows[ii + j, sl] + brows[ii + j, sl] + \
                            cte_t[tj, sl]
                        acc = acc + jnp.maximum(h, 0.0) * w2b[fblk]
                    accs[pl.ds((ii + j) * 16, 16)] = acc

            @pl.loop(0, KL, step=16)
            def _red(ii):
                bi = (ii + lane) * 16
                tot = plsc.load_gather(accs, [bi])
                for cc in range(1, 16):
                    tot = tot + plsc.load_gather(accs, [bi + cc])
                scv[pl.ds(ii, 16)] = tot

            pltpu.sync_copy(scv, sc_out.at[pl.ds(base + c0, KL)])

        _start(0, srcva, dstva, etva, arows_a, brows_a, sema, semia)

        @pl.loop(0, NC // 2)
        def _pair(i):
            c0 = 2 * KL * i
            _start(c0 + KL, srcvb, dstvb, etvb, arows_b, brows_b, semb, semib)
            _wait(srcva, dstva, arows_a, brows_a, sema)
            _process(c0, etva, arows_a, brows_a)

            @pl.when(c0 + 2 * KL < EPC)
            def _():
                _start(c0 + 2 * KL, srcva, dstva, etva, arows_a, brows_a,
                       sema, semia)

            _wait(srcvb, dstvb, arows_b, brows_b, semb)
            _process(c0 + KL, etvb, arows_b, brows_b)

    return k(ea, eb_p, cte, w2, src_p, dst_p, et_p)


def kernel(node_features, edge_index, edge_type, edge_descriptor, query, params):
    p = params
    src = edge_index[0]
    dst = edge_index[1]
    pad = EP - E
    api = jnp.arange(pad, dtype=i32)
    src_p = jnp.concatenate([src, api % N])
    dst_p = jnp.concatenate([dst, N + (api % (NP - N))])
    et_p = jnp.concatenate([edge_type, jnp.zeros((pad,), i32)])

    type_emb = jnp.tanh(edge_descriptor @ p['W_schema'] + p['b_schema'])
    q_emb = query @ p['W_query'] + p['b_query']

    wnode = p['W_node'] * p['tangent_scale']
    bnode = (p['b_node'] * p['tangent_scale']).reshape(1, H)
    t = _prep(node_features, wnode, bnode)

    snaps = []
    aggp = sp = None
    for l in range(L):
        avs = p['attn_a'][l, :H]
        avd = p['attn_a'][l, H:2 * H]
        avt = p['attn_a'][l, 2 * H:]
        a_t = type_emb @ avt
        avmat = jnp.concatenate(
            [avs[:, None], avd[:, None], jnp.zeros((H, 6), f32)], axis=1)
        m, sd = _mm(t, p['W_mp'][l], p['b_mp'][l].reshape(1, H), avmat)
        a_s = sd[:, 0]
        a_d = sd[:, 1]
        cst = jnp.max(a_s) + jnp.max(a_t)
        a_d_p = jnp.concatenate([a_d, jnp.zeros((NP - N,), f32)])
        cst16 = jnp.full((16,), cst, f32)
        aggp, sp = _sc_layer(m, a_s, a_d_p, a_t, cst16, src_p, dst_p, et_p)
        s_col = (sp[0, :N] + sp[1, :N])[:, None]
        if l < L - 1:
            t = _combine(aggp, s_col)
            snaps.append(t)

    wq = (p['rms_w'] * p['depth_q'][L - 1]).reshape(1, H)
    wn1a = p['Wn1'][:H]
    cn = (q_emb @ p['Wn1'][H:] + p['bn1']).reshape(1, H)
    wn2 = p['Wn2'].reshape(1, H)
    we1s = p['We1'][:H]
    we1d = p['We1'][H:2 * H]
    x_out, ns, ea, eb = _final(aggp, s_col, snaps[0], snaps[1], wq, wn1a, cn, wn2,
                               we1s, we1d)
    node_scores = ns[:, 0] + p['bn2'][0]

    eb_p = jnp.concatenate([eb, jnp.zeros((NP - N, H), f32)])
    cte = type_emb @ p['We1'][2 * H:2 * H + TD] + \
        (q_emb @ p['We1'][2 * H + TD:] + p['be1']).reshape(1, H)
    es_p = _sc_edge(ea, eb_p, cte, p['We2'][:, 0], src_p, dst_p, et_p)
    edge_scores = es_p[:E] + p['be2'][0]

    return node_scores, edge_scores, x_out, type_emb


# speedup vs baseline: 1.1326x; 1.1326x over previous
import dataclasses
import functools

import jax
import jax.numpy as jnp
import numpy as np
from jax import lax
from jax.experimental import pallas as pl
from jax.experimental.pallas import tpu as pltpu
from jax.experimental.pallas import tpu_sc as plsc

N = 10000
E = 160000
H = 128
TD = 8
L = 3
T = 16
NP = 10240
EP = 163840
EPC = EP // 32
KL = 160
NC = EPC // KL
ROWS_PER_TILE = NP // 16
CLIP = float(np.arctanh(np.float32(1.0 - 1e-5)))
BN = 2000
GRID = N // BN

f32 = jnp.float32
i32 = jnp.int32


def _clipnorm(v):
    nv = jnp.maximum(jnp.sqrt(jnp.sum(v * v, axis=-1, keepdims=True)), 1e-15)
    return v * jnp.minimum(1.0, CLIP / nv)


def _prep_body(nf_ref, w_ref, b_ref, o_ref):
    v = jnp.dot(nf_ref[...], w_ref[...], preferred_element_type=f32, precision=jax.lax.Precision.HIGHEST) + b_ref[...]
    o_ref[...] = _clipnorm(v)


def _prep(nf, w, b):
    return pl.pallas_call(
        _prep_body,
        grid=(GRID,),
        in_specs=[
            pl.BlockSpec((BN, H), lambda i: (i, 0)),
            pl.BlockSpec((H, H), lambda i: (0, 0)),
            pl.BlockSpec((1, H), lambda i: (0, 0)),
        ],
        out_specs=pl.BlockSpec((BN, H), lambda i: (i, 0)),
        out_shape=jax.ShapeDtypeStruct((N, H), f32),
    )(nf, w, b)


def _mm_body(t_ref, w_ref, b_ref, av_ref, m_ref, sd_ref):
    t = t_ref[...]
    m_ref[...] = jnp.dot(t, w_ref[...], preferred_element_type=f32, precision=jax.lax.Precision.HIGHEST) + b_ref[...]
    sd_ref[...] = jnp.dot(t, av_ref[...], preferred_element_type=f32, precision=jax.lax.Precision.HIGHEST)


def _mm(t, w, b, av):
    return pl.pallas_call(
        _mm_body,
        grid=(GRID,),
        in_specs=[
            pl.BlockSpec((BN, H), lambda i: (i, 0)),
            pl.BlockSpec((H, H), lambda i: (0, 0)),
            pl.BlockSpec((1, H), lambda i: (0, 0)),
            pl.BlockSpec((H, 8), lambda i: (0, 0)),
        ],
        out_specs=[
            pl.BlockSpec((BN, H), lambda i: (i, 0)),
            pl.BlockSpec((BN, 8), lambda i: (i, 0)),
        ],
        out_shape=[
            jax.ShapeDtypeStruct((N, H), f32),
            jax.ShapeDtypeStruct((N, 8), f32),
        ],
    )(t, w, b, av)


def _combine_body(a0_ref, a1_ref, s_ref, o_ref):
    r = 1.0 / (s_ref[...] + 1e-15)
    agg = (a0_ref[0] + a1_ref[0]) * r
    o_ref[...] = _clipnorm(jnp.maximum(agg, 0.0))


def _combine(aggp, s_col):
    return pl.pallas_call(
        _combine_body,
        grid=(GRID,),
        in_specs=[
            pl.BlockSpec((1, BN, H), lambda i: (0, i, 0)),
            pl.BlockSpec((1, BN, H), lambda i: (1, i, 0)),
            pl.BlockSpec((BN, 1), lambda i: (i, 0)),
        ],
        out_specs=pl.BlockSpec((BN, H), lambda i: (i, 0)),
        out_shape=jax.ShapeDtypeStruct((N, H), f32),
    )(aggp, aggp, s_col)


def _final_body(a0_ref, a1_ref, s_ref, s1_ref, s2_ref, wq_ref, wn1_ref, cn_ref,
                wn2_ref, we1s_ref, we1d_ref,
                x_ref, ns_ref, ea_ref, eb_ref):
    r = 1.0 / (s_ref[...] + 1e-15)
    agg = (a0_ref[0] + a1_ref[0]) * r
    v = jnp.maximum(agg, 0.0)
    nv = jnp.maximum(jnp.sqrt(jnp.sum(v * v, axis=-1, keepdims=True)), 1e-15)
    x_ref[...] = jnp.tanh(nv) * v / nv
    s3 = v * jnp.minimum(1.0, CLIP / nv)
    s1 = s1_ref[...]
    s2 = s2_ref[...]
    wq = wq_ref[...]
    eps = 1e-6

    def dlog(sn):
        ms = jnp.sqrt(jnp.mean(sn * sn, axis=-1, keepdims=True) + eps)
        return jnp.sum(sn * wq, axis=-1, keepdims=True) / ms

    d1, d2, d3 = dlog(s1), dlog(s2), dlog(s3)
    dm = jnp.maximum(jnp.maximum(d1, d2), d3)
    e1 = jnp.exp(d1 - dm)
    e2 = jnp.exp(d2 - dm)
    e3 = jnp.exp(d3 - dm)
    tang = (e1 * s1 + e2 * s2 + e3 * s3) / (e1 + e2 + e3)
    hid = jnp.maximum(
        jnp.dot(tang, wn1_ref[...], preferred_element_type=f32, precision=jax.lax.Precision.HIGHEST) + cn_ref[...], 0.0)
    ns_ref[...] = jnp.sum(hid * wn2_ref[...], axis=-1, keepdims=True)
    ea_ref[...] = jnp.dot(tang, we1s_ref[...], preferred_element_type=f32, precision=jax.lax.Precision.HIGHEST)
    eb_ref[...] = jnp.dot(tang, we1d_ref[...], preferred_element_type=f32, precision=jax.lax.Precision.HIGHEST)


def _final(aggp, s_col, s1, s2, wq, wn1a, cn, wn2, we1s, we1d):
    return pl.pallas_call(
        _final_body,
        grid=(GRID,),
        in_specs=[
            pl.BlockSpec((1, BN, H), lambda i: (0, i, 0)),
            pl.BlockSpec((1, BN, H), lambda i: (1, i, 0)),
            pl.BlockSpec((BN, 1), lambda i: (i, 0)),
            pl.BlockSpec((BN, H), lambda i: (i, 0)),
            pl.BlockSpec((BN, H), lambda i: (i, 0)),
            pl.BlockSpec((1, H), lambda i: (0, 0)),
            pl.BlockSpec((H, H), lambda i: (0, 0)),
            pl.BlockSpec((1, H), lambda i: (0, 0)),
            pl.BlockSpec((1, H), lambda i: (0, 0)),
            pl.BlockSpec((H, H), lambda i: (0, 0)),
            pl.BlockSpec((H, H), lambda i: (0, 0)),
        ],
        out_specs=[
            pl.BlockSpec((BN, H), lambda i: (i, 0)),
            pl.BlockSpec((BN, 1), lambda i: (i, 0)),
            pl.BlockSpec((BN, H), lambda i: (i, 0)),
            pl.BlockSpec((BN, H), lambda i: (i, 0)),
        ],
        out_shape=[
            jax.ShapeDtypeStruct((N, H), f32),
            jax.ShapeDtypeStruct((N, 1), f32),
            jax.ShapeDtypeStruct((N, H), f32),
            jax.ShapeDtypeStruct((N, H), f32),
        ],
    )(aggp, aggp, s_col, s1, s2, wq, wn1a, cn, wn2, we1s, we1d)


def _mesh():
    return plsc.VectorSubcoreMesh(core_axis_name="c", subcore_axis_name="s")


def _sc_params():
    cp = pltpu.CompilerParams()
    if "needs_layout_passes" in pltpu.CompilerParams.__dataclass_fields__:
        cp = dataclasses.replace(cp, needs_layout_passes=False)
    return cp


def _sc_layer(m, a_s, a_d_p, a_t, cst16, src_p, dst_p, et_p):
    @functools.partial(
        pl.kernel,
        out_type=[
            jax.ShapeDtypeStruct((2, NP, H), f32),
            jax.ShapeDtypeStruct((2, NP), f32),
        ],
        mesh=_mesh(),
        compiler_params=_sc_params(),
        scratch_types=[
            pltpu.VMEM((16,), f32),
            pltpu.VMEM((16,), f32),
            pltpu.VMEM((KL,), i32),
            pltpu.VMEM((KL,), i32),
            pltpu.VMEM((KL,), i32),
            pltpu.VMEM((KL,), i32),
            pltpu.VMEM((KL,), i32),
            pltpu.VMEM((KL,), i32),
            pltpu.VMEM((KL,), f32),
            pltpu.VMEM((KL,), f32),
            pltpu.VMEM((KL,), f32),
            pltpu.VMEM((KL,), f32),
            pltpu.VMEM((KL,), f32),
            pltpu.VMEM((KL,), f32),
            pltpu.VMEM((KL, H), f32),
            pltpu.VMEM((KL, H), f32),
            pltpu.VMEM_SHARED((NP, H), f32),
            pltpu.VMEM_SHARED((NP,), f32),
            pltpu.SemaphoreType.DMA,
            pltpu.SemaphoreType.DMA,
            pltpu.SemaphoreType.DMA,
            pltpu.SemaphoreType.DMA,
            pltpu.SemaphoreType.DMA,
            pltpu.SemaphoreType.DMA,
        ],
    )
    def k(m_hbm, as_hbm, ad_hbm, at_hbm, c_hbm, src_hbm, dst_hbm, et_hbm,
          agg_out, s_out,
          at_t, cst_t, srcva, dstva, etva, srcvb, dstvb, etvb,
          asga, adga, asgb, adgb, wva, wvb, rows_a, rows_b,
          agg_sh, s_sh, sema, semb, semsa, semsb, semia, semib):
        cid = lax.axis_index("c")
        sid = lax.axis_index("s")
        pltpu.sync_copy(at_hbm, at_t)
        pltpu.sync_copy(c_hbm, cst_t)
        zero = jnp.zeros((16,), f32)

        @pl.loop(0, KL)
        def _zr(rr):
            for fblk in range(8):
                rows_a[rr, pl.ds(fblk * 16, 16)] = zero

        @pl.loop(0, KL, step=16)
        def _zs(ii):
            wva[pl.ds(ii, 16)] = zero

        row0 = sid * ROWS_PER_TILE
        for kk in range(ROWS_PER_TILE // KL):
            pltpu.sync_copy(rows_a, agg_sh.at[pl.ds(row0 + kk * KL, KL)])
            pltpu.sync_copy(wva, s_sh.at[pl.ds(row0 + kk * KL, KL)])
        plsc.subcore_barrier()

        base = cid * (EP // 2) + sid * EPC

        def _start(c0, srcv, dstv, etv, asg, adg, rows, sem, semi):
            off = base + c0
            h1 = pltpu.async_copy(src_hbm.at[pl.ds(off, KL)], srcv, semi)
            h2 = pltpu.async_copy(dst_hbm.at[pl.ds(off, KL)], dstv, semi)
            h3 = pltpu.async_copy(et_hbm.at[pl.ds(off, KL)], etv, semi)
            h1.wait()
            h2.wait()
            h3.wait()
            ga = pltpu.async_copy(as_hbm.at[srcv], asg, sem)
            gb = pltpu.async_copy(ad_hbm.at[dstv], adg, sem)
            gc = pltpu.async_copy(m_hbm.at[srcv], rows, sem)
            return (ga, gb, gc)

        def _wait(hs):
            for h in hs:
                h.wait()

        def _process(dstv, etv, asg, adg, wv, rows):
            cv = cst_t[...]

            @pl.loop(0, KL, step=16)
            def _w(ii):
                a_sv = asg[pl.ds(ii, 16)]
                a_dv = adg[pl.ds(ii, 16)]
                ti = etv[pl.ds(ii, 16)]
                atg = plsc.load_gather(at_t, [ti])
                z = a_sv + a_dv + atg
                lr = jnp.maximum(z, 0.2 * z)
                adc = a_dv + cv
                ov = jnp.maximum(adc, 0.2 * adc)
                wvec = jnp.exp(lr - ov)
                wv[pl.ds(ii, 16)] = wvec
                for j in range(16):
                    w = wvec[j]
                    for fblk in range(8):
                        sl = pl.ds(fblk * 16, 16)
                        rows[ii + j, sl] = rows[ii + j, sl] * w

        def _scatter(dstv, wv, rows, sem):
            h1 = pltpu.async_copy(rows, agg_sh.at[dstv], sem, add=True)
            h2 = pltpu.async_copy(wv, s_sh.at[dstv], sem, add=True)
            return (h1, h2)

        _start(0, srcva, dstva, etva, asga, adga, rows_a, sema, semia)

        @pl.loop(0, NC // 2)
        def _pair(i):
            c0 = 2 * KL * i
            hgb = _start(c0 + KL, srcvb, dstvb, etvb, asgb, adgb, rows_b,
                         semb, semib)
            pltpu.make_async_copy(as_hbm.at[srcva], asga, sema).wait()
            pltpu.make_async_copy(ad_hbm.at[dstva], adga, sema).wait()
            pltpu.make_async_copy(m_hbm.at[srcva], rows_a, sema).wait()
            _process(dstva, etva, asga, adga, wva, rows_a)
            hsa = _scatter(dstva, wva, rows_a, semsa)
            _wait(hgb)
            _process(dstvb, etvb, asgb, adgb, wvb, rows_b)
            _wait(hsa)
            hsb = _scatter(dstvb, wvb, rows_b, semsb)

            @pl.when(c0 + 2 * KL < EPC)
            def _():
                _start(c0 + 2 * KL, srcva, dstva, etva, asga, adga, rows_a,
                       sema, semia)

            _wait(hsb)

        plsc.subcore_barrier()
        sl = pl.ds(row0, ROWS_PER_TILE)
        pltpu.sync_copy(agg_sh.at[sl], agg_out.at[cid, sl])
        pltpu.sync_copy(s_sh.at[sl], s_out.at[cid, sl])

    return k(m, a_s, a_d_p, a_t, cst16, src_p, dst_p, et_p)


def _sc_edge(ea, eb_p, cte, w2, src_p, dst_p, et_p):
    @functools.partial(
        pl.kernel,
        out_type=jax.ShapeDtypeStruct((EP,), f32),
        mesh=_mesh(),
        compiler_params=_sc_params(),
        scratch_types=[
            pltpu.VMEM((T, H), f32),
            pltpu.VMEM((H,), f32),
            pltpu.VMEM((KL,), i32),
            pltpu.VMEM((KL,), i32),
            pltpu.VMEM((KL,), i32),
            pltpu.VMEM((KL,), i32),
            pltpu.VMEM((KL,), i32),
            pltpu.VMEM((KL,), i32),
            pltpu.VMEM((KL, H), f32),
            pltpu.VMEM((KL, H), f32),
            pltpu.VMEM((KL, H), f32),
            pltpu.VMEM((KL, H), f32),
            pltpu.VMEM((KL * 16,), f32),
            pltpu.VMEM((KL,), f32),
            pltpu.SemaphoreType.DMA,
            pltpu.SemaphoreType.DMA,
            pltpu.SemaphoreType.DMA,
            pltpu.SemaphoreType.DMA,
        ],
    )
    def k(ea_hbm, eb_hbm, cte_hbm, w2_hbm, src_hbm, dst_hbm, et_hbm, sc_out,
          cte_t, w2_t, srcva, dstva, etva, srcvb, dstvb, etvb,
          arows_a, brows_a, arows_b, brows_b, accs, scv, sema, semb,
          semia, semib):
        cid = lax.axis_index("c")
        sid = lax.axis_index("s")
        pltpu.sync_copy(cte_hbm, cte_t)
        pltpu.sync_copy(w2_hbm, w2_t)
        base = cid * (EP // 2) + sid * EPC
        lane = lax.iota(i32, 16)
        w2b = [w2_t[pl.ds(fblk * 16, 16)] for fblk in range(8)]

        def _start(c0, srcv, dstv, etv, arows, brows, sem, semi):
            off = base + c0
            h1 = pltpu.async_copy(src_hbm.at[pl.ds(off, KL)], srcv, semi)
            h2 = pltpu.async_copy(dst_hbm.at[pl.ds(off, KL)], dstv, semi)
            h3 = pltpu.async_copy(et_hbm.at[pl.ds(off, KL)], etv, semi)
            h1.wait()
            h2.wait()
            h3.wait()
            pltpu.async_copy(ea_hbm.at[srcv], arows, sem)
            pltpu.async_copy(eb_hbm.at[dstv], brows, sem)

        def _wait(srcv, dstv, arows, brows, sem):
            pltpu.make_async_copy(ea_hbm.at[srcv], arows, sem).wait()
            pltpu.make_async_copy(eb_hbm.at[dstv], brows, sem).wait()

        def _process(c0, etv, arows, brows):
            @pl.loop(0, KL, step=16)
            def _edge(ii):
                tvec = etv[pl.ds(ii, 16)]
                for j in range(16):
                    tj = tvec[j]
                    acc = jnp.zeros((16,), f32)
                    for fblk in range(8):
                        sl = pl.ds(fblk * 16, 16)
                        h = arows[ii + j, sl] + brows[ii + j, sl] + \
                            cte_t[tj, sl]
                        acc = acc + jnp.maximum(h, 0.0) * w2b[fblk]
                    accs[pl.ds((ii + j) * 16, 16)] = acc

            @pl.loop(0, KL, step=16)
            def _red(ii):
                bi = (ii + lane) * 16
                tot = plsc.load_gather(accs, [bi])
                for cc in range(1, 16):
                    tot = tot + plsc.load_gather(accs, [bi + cc])
                scv[pl.ds(ii, 16)] = tot

            pltpu.sync_copy(scv, sc_out.at[pl.ds(base + c0, KL)])

        _start(0, srcva, dstva, etva, arows_a, brows_a, sema, semia)

        @pl.loop(0, NC // 2)
        def _pair(i):
            c0 = 2 * KL * i
            _start(c0 + KL, srcvb, dstvb, etvb, arows_b, brows_b, semb, semib)
            _wait(srcva, dstva, arows_a, brows_a, sema)
            _process(c0, etva, arows_a, brows_a)

            @pl.when(c0 + 2 * KL < EPC)
            def _():
                _start(c0 + 2 * KL, srcva, dstva, etva, arows_a, brows_a,
                       sema, semia)

            _wait(srcvb, dstvb, arows_b, brows_b, semb)
            _process(c0 + KL, etvb, arows_b, brows_b)

    return k(ea, eb_p, cte, w2, src_p, dst_p, et_p)


def kernel(node_features, edge_index, edge_type, edge_descriptor, query, params):
    p = params
    src = edge_index[0]
    dst = edge_index[1]
    pad = EP - E
    api = jnp.arange(pad, dtype=i32)
    src_p = jnp.concatenate([src, api % N])
    dst_p = jnp.concatenate([dst, N + (api % (NP - N))])
    et_p = jnp.concatenate([edge_type, jnp.zeros((pad,), i32)])

    type_emb = jnp.tanh(edge_descriptor @ p['W_schema'] + p['b_schema'])
    q_emb = query @ p['W_query'] + p['b_query']

    wnode = p['W_node'] * p['tangent_scale']
    bnode = (p['b_node'] * p['tangent_scale']).reshape(1, H)
    t = _prep(node_features, wnode, bnode)

    snaps = []
    aggp = sp = None
    for l in range(L):
        avs = p['attn_a'][l, :H]
        avd = p['attn_a'][l, H:2 * H]
        avt = p['attn_a'][l, 2 * H:]
        a_t = type_emb @ avt
        avmat = jnp.concatenate(
            [avs[:, None], avd[:, None], jnp.zeros((H, 6), f32)], axis=1)
        m, sd = _mm(t, p['W_mp'][l], p['b_mp'][l].reshape(1, H), avmat)
        a_s = sd[:, 0]
        a_d = sd[:, 1]
        cst = jnp.max(a_s) + jnp.max(a_t)
        a_d_p = jnp.concatenate([a_d, jnp.zeros((NP - N,), f32)])
        cst16 = jnp.full((16,), cst, f32)
        aggp, sp = _sc_layer(m, a_s, a_d_p, a_t, cst16, src_p, dst_p, et_p)
        s_col = (sp[0, :N] + sp[1, :N])[:, None]
        if l < L - 1:
            t = _combine(aggp, s_col)
            snaps.append(t)

    wq = (p['rms_w'] * p['depth_q'][L - 1]).reshape(1, H)
    wn1a = p['Wn1'][:H]
    cn = (q_emb @ p['Wn1'][H:] + p['bn1']).reshape(1, H)
    wn2 = p['Wn2'].reshape(1, H)
    we1s = p['We1'][:H]
    we1d = p['We1'][H:2 * H]
    x_out, ns, ea, eb = _final(aggp, s_col, snaps[0], snaps[1], wq, wn1a, cn, wn2,
                               we1s, we1d)
    node_scores = ns[:, 0] + p['bn2'][0]

    eb_p = jnp.concatenate([eb, jnp.zeros((NP - N, H), f32)])
    cte = type_emb @ p['We1'][2 * H:2 * H + TD] + \
        (q_emb @ p['We1'][2 * H + TD:] + p['be1']).reshape(1, H)
    es_p = _sc_edge(ea, eb_p, cte, p['We2'][:, 0], src_p, dst_p, et_p)
    edge_scores = es_p[:E] + p['be2'][0]

    return node_scores, edge_scores, x_out, type_emb
